# baseline (speedup 1.0000x reference)
import functools

import jax
import jax.numpy as jnp
from jax import lax
from jax.experimental import pallas as pl
from jax.experimental.pallas import tpu as pltpu
from jax.experimental.pallas import tpu_sc as plsc

EMBED = 64
SCALE = 8.0
LANES = 16
NC = 2
NS = 16
NW = NC * NS
G = 128
NBUF = 2


@functools.lru_cache(maxsize=None)
def _build_pack(vocab: int):
    nblk = (vocab + 2 * EMBED - 1) // (2 * EMBED)

    def body(tt_ref, out_ref):
        out_ref[:, 0:EMBED] = tt_ref[...].T * SCALE

    return pl.pallas_call(
        body,
        grid=(nblk,),
        in_specs=[pl.BlockSpec((EMBED, 2 * EMBED), lambda c: (0, c))],
        out_specs=pl.BlockSpec((2 * EMBED, 2 * EMBED), lambda c: (c, 0)),
        out_shape=jax.ShapeDtypeStruct((vocab, 2 * EMBED), jnp.float32),
    )


@functools.lru_cache(maxsize=None)
def _build(batch: int, hist: int, vocab: int):
    n_total = batch * hist
    assert n_total % (NW * G) == 0 and batch % G == 0
    n_g = n_total // (NW * G)
    cpb = batch // G

    mesh = plsc.VectorSubcoreMesh(core_axis_name="c", subcore_axis_name="s")

    @functools.partial(
        pl.kernel,
        mesh=mesh,
        out_type=jax.ShapeDtypeStruct((hist, EMBED, batch), jnp.float32),
        scratch_types=[
            pltpu.VMEM((n_g, G), jnp.int32),
            pltpu.VMEM((NBUF, G, 2 * EMBED), jnp.float32),
            pltpu.VMEM((NBUF, EMBED, G), jnp.float32),
            pltpu.SemaphoreType.DMA((NBUF,)),
            pltpu.SemaphoreType.DMA((NBUF,)),
        ],
        compiler_params=pltpu.CompilerParams(
            use_tc_tiling_on_sc=True, needs_layout_passes=False
        ),
    )
    def emb_kernel(xg_hbm, tpad_hbm, out_hbm, idx_v, lbuf, sbuf, gsem, ssem):
        wid = lax.axis_index("s") * NC + lax.axis_index("c")
        g0 = wid * n_g
        pltpu.sync_copy(xg_hbm.at[pl.ds(g0, n_g)], idx_v)

        def start_gather(gl, p):
            pltpu.make_async_copy(
                tpad_hbm.at[idx_v.at[gl]], lbuf.at[p], gsem.at[p]
            ).start()

        def wait_gather(gl, p):
            pltpu.make_async_copy(
                tpad_hbm.at[idx_v.at[gl]], lbuf.at[p], gsem.at[p]
            ).wait()

        def out_slice(gl):
            g = g0 + gl
            h = g // cpb
            b0 = (g % cpb) * G
            return out_hbm.at[h, :, pl.ds(b0, G)]

        def start_scatter(gl, p):
            pltpu.make_async_copy(sbuf.at[p], out_slice(gl), ssem.at[p]).start()

        def wait_scatter(gl, p):
            pltpu.make_async_copy(sbuf.at[p], out_slice(gl), ssem.at[p]).wait()

        lane = lax.iota(jnp.int32, LANES)
        row_ids = [lane + (q * LANES) for q in range(G // LANES)]
        rot = [(lane + d) & (LANES - 1) for d in range(LANES)]

        for p in range(NBUF):
            start_gather(p, p)

        def block(gl, carry):
            p = lax.rem(gl, NBUF)
            wait_gather(gl, p)

            @pl.when(gl >= NBUF)
            def _():
                wait_scatter(gl - NBUF, p)

            @plsc.parallel_loop(0, EMBED // LANES, step=1, unroll=1)
            def etile(e0t):
                e0b = jnp.full((LANES,), e0t * LANES, jnp.int32)
                rcs = [e0b + rot[d] for d in range(LANES)]
                for q in range(G // LANES):
                    for d in range(LANES):
                        v = plsc.load_gather(
                            lbuf.at[p], [row_ids[q], rcs[d]]
                        )
                        plsc.store_scatter(
                            sbuf.at[p], [rcs[d], row_ids[q]], v
                        )

            @pl.when(gl + NBUF < n_g)
            def _():
                start_gather(gl + NBUF, p)

            start_scatter(gl, p)
            return carry

        lax.fori_loop(0, n_g, block, 0)

        for p in range(NBUF):
            wait_scatter(n_g - NBUF + p, p)

    return emb_kernel


def kernel(x, table):
    b, h = x.shape
    vocab, d = table.shape
    xg = x.T.reshape(b * h // G, G).astype(jnp.int32)
    tpad = _build_pack(vocab)(table.T)
    out_t = _build(b, h, vocab)(xg, tpad)
    return out_t.transpose(2, 0, 1)


# speedup vs baseline: 5.4797x; 5.4797x over previous
import functools

import jax
import jax.numpy as jnp
from jax import lax
from jax.experimental import pallas as pl
from jax.experimental.pallas import tpu as pltpu
from jax.experimental.pallas import tpu_sc as plsc

EMBED = 64
SCALE = 8.0
LANES = 16
NC = 2
NS = 16
NW = NC * NS
G = 128
NBUF = 2


@functools.lru_cache(maxsize=None)
def _build_pack(vocab: int):
    w = 2048
    nblk = (vocab + w - 1) // w

    def body(tt_ref, out_ref):
        out_ref[:, 0:EMBED] = tt_ref[...].T * SCALE

    return pl.pallas_call(
        body,
        grid=(nblk,),
        in_specs=[pl.BlockSpec((EMBED, w), lambda c: (0, c))],
        out_specs=pl.BlockSpec((w, 2 * EMBED), lambda c: (c, 0)),
        out_shape=jax.ShapeDtypeStruct((vocab, 2 * EMBED), jnp.float32),
    )


@functools.lru_cache(maxsize=None)
def _build(batch: int, hist: int, vocab: int):
    n_total = batch * hist
    assert n_total % (NW * G) == 0 and batch % G == 0
    n_g = n_total // (NW * G)
    cpb = batch // G

    mesh = plsc.VectorSubcoreMesh(core_axis_name="c", subcore_axis_name="s")

    @functools.partial(
        pl.kernel,
        mesh=mesh,
        out_type=jax.ShapeDtypeStruct((hist, EMBED, batch), jnp.float32),
        scratch_types=[
            pltpu.VMEM((n_g, G), jnp.int32),
            pltpu.VMEM((NBUF, G, 2 * EMBED), jnp.float32),
            pltpu.VMEM((NBUF, EMBED, G), jnp.float32),
            pltpu.SemaphoreType.DMA((NBUF,)),
            pltpu.SemaphoreType.DMA((NBUF,)),
        ],
        compiler_params=pltpu.CompilerParams(
            use_tc_tiling_on_sc=True, needs_layout_passes=False
        ),
    )
    def emb_kernel(xg_hbm, tpad_hbm, out_hbm, idx_v, lbuf, sbuf, gsem, ssem):
        wid = lax.axis_index("s") * NC + lax.axis_index("c")
        g0 = wid * n_g
        pltpu.sync_copy(xg_hbm.at[pl.ds(g0, n_g)], idx_v)

        def start_gather(gl, p):
            pltpu.make_async_copy(
                tpad_hbm.at[idx_v.at[gl]], lbuf.at[p], gsem.at[p]
            ).start()

        def wait_gather(gl, p):
            pltpu.make_async_copy(
                tpad_hbm.at[idx_v.at[gl]], lbuf.at[p], gsem.at[p]
            ).wait()

        def out_slice(gl):
            g = g0 + gl
            h = g // cpb
            b0 = (g % cpb) * G
            return out_hbm.at[h, :, pl.ds(b0, G)]

        def start_scatter(gl, p):
            pltpu.make_async_copy(sbuf.at[p], out_slice(gl), ssem.at[p]).start()

        def wait_scatter(gl, p):
            pltpu.make_async_copy(sbuf.at[p], out_slice(gl), ssem.at[p]).wait()

        lane = lax.iota(jnp.int32, LANES)
        row_ids = [lane + (q * LANES) for q in range(G // LANES)]
        rot = [(lane + d) & (LANES - 1) for d in range(LANES)]

        for p in range(NBUF):
            start_gather(p, p)

        def block(gl, carry):
            p = lax.rem(gl, NBUF)
            wait_gather(gl, p)

            @pl.when(gl >= NBUF)
            def _():
                wait_scatter(gl - NBUF, p)

            @plsc.parallel_loop(0, EMBED // LANES, step=1, unroll=1)
            def etile(e0t):
                e0b = jnp.full((LANES,), e0t * LANES, jnp.int32)
                rcs = [e0b + rot[d] for d in range(LANES)]
                for q in range(G // LANES):
                    for d in range(LANES):
                        v = plsc.load_gather(
                            lbuf.at[p], [row_ids[q], rcs[d]]
                        )
                        plsc.store_scatter(
                            sbuf.at[p], [rcs[d], row_ids[q]], v
                        )

            @pl.when(gl + NBUF < n_g)
            def _():
                start_gather(gl + NBUF, p)

            start_scatter(gl, p)
            return carry

        lax.fori_loop(0, n_g, block, 0)

        for p in range(NBUF):
            wait_scatter(n_g - NBUF + p, p)

    return emb_kernel


def kernel(x, table):
    b, h = x.shape
    vocab, d = table.shape
    xg = x.T.reshape(b * h // G, G).astype(jnp.int32)
    tpad = _build_pack(vocab)(table.T)
    out_t = _build(b, h, vocab)(xg, tpad)
    return out_t.transpose(2, 0, 1)


# speedup vs baseline: 7.2547x; 1.3239x over previous
import functools

import jax
import jax.numpy as jnp
from jax import lax
from jax.experimental import pallas as pl
from jax.experimental.pallas import tpu as pltpu
from jax.experimental.pallas import tpu_sc as plsc

EMBED = 64
SCALE = 8.0
LANES = 16
NC = 2
NS = 16
NW = NC * NS
G = 128
NBUF = 2


@functools.lru_cache(maxsize=None)
def _build_pack(vocab: int):
    w = 8192
    nblk = (vocab + w - 1) // w

    def body(tt_ref, out_ref):
        out_ref[:, 0:EMBED] = tt_ref[...].T * SCALE

    return pl.pallas_call(
        body,
        grid=(nblk,),
        in_specs=[pl.BlockSpec((EMBED, w), lambda c: (0, c))],
        out_specs=pl.BlockSpec((w, 2 * EMBED), lambda c: (c, 0)),
        out_shape=jax.ShapeDtypeStruct((vocab, 2 * EMBED), jnp.float32),
    )


@functools.lru_cache(maxsize=None)
def _build(batch: int, hist: int, vocab: int):
    n_total = batch * hist
    assert n_total % (NW * G) == 0 and batch % G == 0
    n_g = n_total // (NW * G)
    cpb = batch // G

    mesh = plsc.VectorSubcoreMesh(core_axis_name="c", subcore_axis_name="s")

    @functools.partial(
        pl.kernel,
        mesh=mesh,
        out_type=jax.ShapeDtypeStruct((hist, EMBED, batch), jnp.float32),
        scratch_types=[
            pltpu.VMEM((n_g, G), jnp.int32),
            pltpu.VMEM((NBUF, G, 2 * EMBED), jnp.float32),
            pltpu.VMEM((NBUF, EMBED, G), jnp.float32),
            pltpu.SemaphoreType.DMA((NBUF,)),
            pltpu.SemaphoreType.DMA((NBUF,)),
        ],
        compiler_params=pltpu.CompilerParams(
            use_tc_tiling_on_sc=True, needs_layout_passes=False
        ),
    )
    def emb_kernel(xg_hbm, tpad_hbm, out_hbm, idx_v, lbuf, sbuf, gsem, ssem):
        wid = lax.axis_index("s") * NC + lax.axis_index("c")
        g0 = wid * n_g
        pltpu.sync_copy(xg_hbm.at[pl.ds(g0, n_g)], idx_v)

        def start_gather(gl, p):
            pltpu.make_async_copy(
                tpad_hbm.at[idx_v.at[gl]], lbuf.at[p], gsem.at[p]
            ).start()

        def wait_gather(gl, p):
            pltpu.make_async_copy(
                tpad_hbm.at[idx_v.at[gl]], lbuf.at[p], gsem.at[p]
            ).wait()

        def out_slice(gl):
            g = g0 + gl
            h = g // cpb
            b0 = (g % cpb) * G
            return out_hbm.at[h, :, pl.ds(b0, G)]

        def start_scatter(gl, p):
            pltpu.make_async_copy(sbuf.at[p], out_slice(gl), ssem.at[p]).start()

        def wait_scatter(gl, p):
            pltpu.make_async_copy(sbuf.at[p], out_slice(gl), ssem.at[p]).wait()

        lane = lax.iota(jnp.int32, LANES)
        row_ids = [lane + (q * LANES) for q in range(G // LANES)]
        rot = [(lane + d) & (LANES - 1) for d in range(LANES)]

        for p in range(NBUF):
            start_gather(p, p)

        def block(gl, carry):
            p = lax.rem(gl, NBUF)
            wait_gather(gl, p)

            @pl.when(gl >= NBUF)
            def _():
                wait_scatter(gl - NBUF, p)

            @plsc.parallel_loop(0, EMBED // LANES, step=1, unroll=1)
            def etile(e0t):
                e0b = jnp.full((LANES,), e0t * LANES, jnp.int32)
                rcs = [e0b + rot[d] for d in range(LANES)]
                for q in range(G // LANES):
                    for d in range(LANES):
                        v = plsc.load_gather(
                            lbuf.at[p], [row_ids[q], rcs[d]]
                        )
                        plsc.store_scatter(
                            sbuf.at[p], [rcs[d], row_ids[q]], v
                        )

            @pl.when(gl + NBUF < n_g)
            def _():
                start_gather(gl + NBUF, p)

            start_scatter(gl, p)
            return carry

        lax.fori_loop(0, n_g, block, 0)

        for p in range(NBUF):
            wait_scatter(n_g - NBUF + p, p)

    return emb_kernel


def kernel(x, table):
    b, h = x.shape
    vocab, d = table.shape
    xg = x.T.reshape(b * h // G, G).astype(jnp.int32)
    tpad = _build_pack(vocab)(table.T)
    out_t = _build(b, h, vocab)(xg, tpad)
    return out_t.transpose(2, 0, 1)


# speedup vs baseline: 7.5147x; 1.0358x over previous
import functools

import jax
import jax.numpy as jnp
from jax import lax
from jax.experimental import pallas as pl
from jax.experimental.pallas import tpu as pltpu
from jax.experimental.pallas import tpu_sc as plsc

EMBED = 64
SCALE = 8.0
LANES = 16
NC = 2
NS = 16
NW = NC * NS
G = 128
NBUF = 2


@functools.lru_cache(maxsize=None)
def _build_pack(vocab: int):
    w = 16384
    nblk = (vocab + w - 1) // w

    def body(tt_ref, out_ref):
        out_ref[:, 0:EMBED] = tt_ref[...].T * SCALE

    return pl.pallas_call(
        body,
        grid=(nblk,),
        in_specs=[pl.BlockSpec((EMBED, w), lambda c: (0, c))],
        out_specs=pl.BlockSpec((w, 2 * EMBED), lambda c: (c, 0)),
        out_shape=jax.ShapeDtypeStruct((vocab, 2 * EMBED), jnp.float32),
    )


@functools.lru_cache(maxsize=None)
def _build(batch: int, hist: int, vocab: int):
    n_total = batch * hist
    assert n_total % (NW * G) == 0 and batch % G == 0
    n_g = n_total // (NW * G)
    cpb = batch // G

    mesh = plsc.VectorSubcoreMesh(core_axis_name="c", subcore_axis_name="s")

    @functools.partial(
        pl.kernel,
        mesh=mesh,
        out_type=jax.ShapeDtypeStruct((hist, EMBED, batch), jnp.float32),
        scratch_types=[
            pltpu.VMEM((n_g, G), jnp.int32),
            pltpu.VMEM((NBUF, G, 2 * EMBED), jnp.float32),
            pltpu.VMEM((NBUF, EMBED, G), jnp.float32),
            pltpu.SemaphoreType.DMA((NBUF,)),
            pltpu.SemaphoreType.DMA((NBUF,)),
        ],
        compiler_params=pltpu.CompilerParams(
            use_tc_tiling_on_sc=True, needs_layout_passes=False
        ),
    )
    def emb_kernel(xg_hbm, tpad_hbm, out_hbm, idx_v, lbuf, sbuf, gsem, ssem):
        wid = lax.axis_index("s") * NC + lax.axis_index("c")
        g0 = wid * n_g
        pltpu.sync_copy(xg_hbm.at[pl.ds(g0, n_g)], idx_v)

        def start_gather(gl, p):
            pltpu.make_async_copy(
                tpad_hbm.at[idx_v.at[gl]], lbuf.at[p], gsem.at[p]
            ).start()

        def wait_gather(gl, p):
            pltpu.make_async_copy(
                tpad_hbm.at[idx_v.at[gl]], lbuf.at[p], gsem.at[p]
            ).wait()

        def out_slice(gl):
            g = g0 + gl
            h = g // cpb
            b0 = (g % cpb) * G
            return out_hbm.at[h, :, pl.ds(b0, G)]

        def start_scatter(gl, p):
            pltpu.make_async_copy(sbuf.at[p], out_slice(gl), ssem.at[p]).start()

        def wait_scatter(gl, p):
            pltpu.make_async_copy(sbuf.at[p], out_slice(gl), ssem.at[p]).wait()

        lane = lax.iota(jnp.int32, LANES)
        row_ids = [lane + (q * LANES) for q in range(G // LANES)]
        rot = [(lane + d) & (LANES - 1) for d in range(LANES)]

        for p in range(NBUF):
            start_gather(p, p)

        def block(gl, carry):
            p = lax.rem(gl, NBUF)
            wait_gather(gl, p)

            @pl.when(gl >= NBUF)
            def _():
                wait_scatter(gl - NBUF, p)

            @plsc.parallel_loop(0, EMBED // LANES, step=1, unroll=1)
            def etile(e0t):
                e0b = jnp.full((LANES,), e0t * LANES, jnp.int32)
                rcs = [e0b + rot[d] for d in range(LANES)]
                for q in range(G // LANES):
                    for d in range(LANES):
                        v = plsc.load_gather(
                            lbuf.at[p], [row_ids[q], rcs[d]]
                        )
                        plsc.store_scatter(
                            sbuf.at[p], [rcs[d], row_ids[q]], v
                        )

            @pl.when(gl + NBUF < n_g)
            def _():
                start_gather(gl + NBUF, p)

            start_scatter(gl, p)
            return carry

        lax.fori_loop(0, n_g, block, 0)

        for p in range(NBUF):
            wait_scatter(n_g - NBUF + p, p)

    return emb_kernel


def kernel(x, table):
    b, h = x.shape
    vocab, d = table.shape
    xg = x.T.reshape(b * h // G, G).astype(jnp.int32)
    tpad = _build_pack(vocab)(table.T)
    out_t = _build(b, h, vocab)(xg, tpad)
    return out_t.transpose(2, 0, 1)


# speedup vs baseline: 7.6379x; 1.0164x over previous
import functools

import jax
import jax.numpy as jnp
from jax import lax
from jax.experimental import pallas as pl
from jax.experimental.pallas import tpu as pltpu
from jax.experimental.pallas import tpu_sc as plsc

EMBED = 64
SCALE = 8.0
LANES = 16
NC = 2
NS = 16
NW = NC * NS
G = 128
NBUF = 3


@functools.lru_cache(maxsize=None)
def _build_pack(vocab: int):
    w = 16384
    nblk = (vocab + w - 1) // w

    def body(tt_ref, out_ref):
        out_ref[:, 0:EMBED] = tt_ref[...].T * SCALE

    return pl.pallas_call(
        body,
        grid=(nblk,),
        in_specs=[pl.BlockSpec((EMBED, w), lambda c: (0, c))],
        out_specs=pl.BlockSpec((w, 2 * EMBED), lambda c: (c, 0)),
        out_shape=jax.ShapeDtypeStruct((vocab, 2 * EMBED), jnp.float32),
    )


@functools.lru_cache(maxsize=None)
def _build(batch: int, hist: int, vocab: int):
    n_total = batch * hist
    assert n_total % (NW * G) == 0 and batch % G == 0
    n_g = n_total // (NW * G)
    cpb = batch // G

    mesh = plsc.VectorSubcoreMesh(core_axis_name="c", subcore_axis_name="s")

    @functools.partial(
        pl.kernel,
        mesh=mesh,
        out_type=jax.ShapeDtypeStruct((hist, EMBED, batch), jnp.float32),
        scratch_types=[
            pltpu.VMEM((n_g, G), jnp.int32),
            pltpu.VMEM((NBUF, G, 2 * EMBED), jnp.float32),
            pltpu.VMEM((NBUF, EMBED, G), jnp.float32),
            pltpu.SemaphoreType.DMA((NBUF,)),
            pltpu.SemaphoreType.DMA((NBUF,)),
        ],
        compiler_params=pltpu.CompilerParams(
            use_tc_tiling_on_sc=True, needs_layout_passes=False
        ),
    )
    def emb_kernel(xg_hbm, tpad_hbm, out_hbm, idx_v, lbuf, sbuf, gsem, ssem):
        wid = lax.axis_index("s") * NC + lax.axis_index("c")
        g0 = wid * n_g
        pltpu.sync_copy(xg_hbm.at[pl.ds(g0, n_g)], idx_v)

        def start_gather(gl, p):
            pltpu.make_async_copy(
                tpad_hbm.at[idx_v.at[gl]], lbuf.at[p], gsem.at[p]
            ).start()

        def wait_gather(gl, p):
            pltpu.make_async_copy(
                tpad_hbm.at[idx_v.at[gl]], lbuf.at[p], gsem.at[p]
            ).wait()

        def out_slice(gl):
            g = g0 + gl
            h = g // cpb
            b0 = (g % cpb) * G
            return out_hbm.at[h, :, pl.ds(b0, G)]

        def start_scatter(gl, p):
            pltpu.make_async_copy(sbuf.at[p], out_slice(gl), ssem.at[p]).start()

        def wait_scatter(gl, p):
            pltpu.make_async_copy(sbuf.at[p], out_slice(gl), ssem.at[p]).wait()

        lane = lax.iota(jnp.int32, LANES)
        row_ids = [lane + (q * LANES) for q in range(G // LANES)]
        rot = [(lane + d) & (LANES - 1) for d in range(LANES)]

        for p in range(NBUF):
            start_gather(p, p)

        def block(gl, carry):
            p = lax.rem(gl, NBUF)
            wait_gather(gl, p)

            @pl.when(gl >= NBUF)
            def _():
                wait_scatter(gl - NBUF, p)

            @plsc.parallel_loop(0, EMBED // LANES, step=1, unroll=1)
            def etile(e0t):
                e0b = jnp.full((LANES,), e0t * LANES, jnp.int32)
                rcs = [e0b + rot[d] for d in range(LANES)]
                for q in range(G // LANES):
                    for d in range(LANES):
                        v = plsc.load_gather(
                            lbuf.at[p], [row_ids[q], rcs[d]]
                        )
                        plsc.store_scatter(
                            sbuf.at[p], [rcs[d], row_ids[q]], v
                        )

            @pl.when(gl + NBUF < n_g)
            def _():
                start_gather(gl + NBUF, p)

            start_scatter(gl, p)
            return carry

        lax.fori_loop(0, n_g, block, 0)

        for p in range(NBUF):
            wait_scatter(n_g - NBUF + p, p)

    return emb_kernel


def kernel(x, table):
    b, h = x.shape
    vocab, d = table.shape
    xg = x.T.reshape(b * h // G, G).astype(jnp.int32)
    tpad = _build_pack(vocab)(table.T)
    out_t = _build(b, h, vocab)(xg, tpad)
    return out_t.transpose(2, 0, 1)


# speedup vs baseline: 9.0253x; 1.1816x over previous
import functools

import jax
import jax.numpy as jnp
from jax import lax
from jax.experimental import pallas as pl
from jax.experimental.pallas import tpu as pltpu
from jax.experimental.pallas import tpu_sc as plsc

EMBED = 64
SCALE = 8.0
LANES = 16
NC = 2
NS = 16
NW = NC * NS
G = 128
NBUF = 3


@functools.lru_cache(maxsize=None)
def _build_pack(vocab: int):
    w = 16384
    nblk = (vocab + w - 1) // w

    def body(tt_ref, out_ref):
        out_ref[:, 0:EMBED] = tt_ref[...].T * SCALE

    return pl.pallas_call(
        body,
        grid=(nblk,),
        in_specs=[pl.BlockSpec((EMBED, w), lambda c: (0, c))],
        out_specs=pl.BlockSpec((w, 2 * EMBED), lambda c: (c, 0)),
        out_shape=jax.ShapeDtypeStruct((vocab, 2 * EMBED), jnp.float32),
    )


@functools.lru_cache(maxsize=None)
def _build(batch: int, hist: int, vocab: int):
    n_total = batch * hist
    assert n_total % (NW * G) == 0 and batch % G == 0
    n_g = n_total // (NW * G)
    cpb = batch // G

    mesh = plsc.VectorSubcoreMesh(core_axis_name="c", subcore_axis_name="s")

    @functools.partial(
        pl.kernel,
        mesh=mesh,
        out_type=jax.ShapeDtypeStruct((hist, EMBED, batch), jnp.float32),
        scratch_types=[
            pltpu.VMEM((n_g, G), jnp.int32),
            pltpu.VMEM((NBUF, G, 2 * EMBED), jnp.float32),
            pltpu.VMEM((NBUF, EMBED, G), jnp.float32),
            pltpu.SemaphoreType.DMA((NBUF,)),
            pltpu.SemaphoreType.DMA((NBUF,)),
        ],
        compiler_params=pltpu.CompilerParams(
            use_tc_tiling_on_sc=True, needs_layout_passes=False
        ),
    )
    def emb_kernel(xg_hbm, tpad_hbm, out_hbm, idx_v, lbuf, sbuf, gsem, ssem):
        wid = lax.axis_index("s") * NC + lax.axis_index("c")
        g0 = wid * n_g
        pltpu.sync_copy(xg_hbm.at[pl.ds(g0, n_g)], idx_v)

        def start_gather(gl, p):
            pltpu.make_async_copy(
                tpad_hbm.at[idx_v.at[gl]], lbuf.at[p], gsem.at[p]
            ).start()

        def wait_gather(gl, p):
            pltpu.make_async_copy(
                tpad_hbm.at[idx_v.at[gl]], lbuf.at[p], gsem.at[p]
            ).wait()

        def out_slice(gl):
            g = g0 + gl
            h = g // cpb
            b0 = (g % cpb) * G
            return out_hbm.at[h, :, pl.ds(b0, G)]

        def start_scatter(gl, p):
            pltpu.make_async_copy(sbuf.at[p], out_slice(gl), ssem.at[p]).start()

        def wait_scatter(gl, p):
            pltpu.make_async_copy(sbuf.at[p], out_slice(gl), ssem.at[p]).wait()

        lane = lax.iota(jnp.int32, LANES)
        row_ids = [lane + (q * LANES) for q in range(G // LANES)]
        rot = [(lane + d) & (LANES - 1) for d in range(LANES)]

        for p in range(NBUF):
            start_gather(p, p)

        def block(gl, carry):
            p = lax.rem(gl, NBUF)
            wait_gather(gl, p)

            @pl.when(gl >= NBUF)
            def _():
                wait_scatter(gl - NBUF, p)

            @plsc.parallel_loop(0, EMBED // LANES, step=1, unroll=2)
            def etile(e0t):
                e0b = jnp.full((LANES,), e0t * LANES, jnp.int32)
                rcs = [e0b + rot[d] for d in range(LANES)]
                for q in range(G // LANES):
                    for d in range(LANES):
                        v = plsc.load_gather(
                            lbuf.at[p], [row_ids[q], rcs[d]]
                        )
                        plsc.store_scatter(
                            sbuf.at[p], [rcs[d], row_ids[q]], v
                        )

            @pl.when(gl + NBUF < n_g)
            def _():
                start_gather(gl + NBUF, p)

            start_scatter(gl, p)
            return carry

        lax.fori_loop(0, n_g, block, 0)

        for p in range(NBUF):
            wait_scatter(n_g - NBUF + p, p)

    return emb_kernel


def kernel(x, table):
    b, h = x.shape
    vocab, d = table.shape
    xg = x.T.reshape(b * h // G, G).astype(jnp.int32)
    tpad = _build_pack(vocab)(table.T)
    out_t = _build(b, h, vocab)(xg, tpad)
    return out_t.transpose(2, 0, 1)
